# baseline (speedup 1.0000x reference)
import functools

import jax
import jax.numpy as jnp
from jax import lax
from jax.experimental import pallas as pl
from jax.experimental.pallas import tpu as pltpu
from jax.experimental.pallas import tpu_sc as plsc

T = 2048
D = 2048
DFF = 1408
E = 8
NGROUP = 4
B = 128
CAP = 2 * T + E * B
NRB = CAP // B
SCALE = 1.0


def _router_body(x_ref, gw_ref, p1_ref, p2_ref, w1_ref, w2_ref, bmap_ref):
    logits = lax.dot_general(gw_ref[...], x_ref[...], (((1,), (1,)), ((), ())),
                             preferred_element_type=jnp.float32)
    m = jnp.max(logits, axis=0, keepdims=True)
    ex = jnp.exp(logits - m)
    sc = ex / jnp.sum(ex, axis=0, keepdims=True)

    rows = [sc[e:e + 1, :] for e in range(E)]
    gm = [jnp.maximum(rows[2 * g], rows[2 * g + 1]) for g in range(NGROUP)]
    i1 = jnp.zeros((1, T), jnp.int32)
    v1 = gm[0]
    for g in range(1, NGROUP):
        take = gm[g] > v1
        i1 = jnp.where(take, g, i1)
        v1 = jnp.where(take, gm[g], v1)
    gm2 = [jnp.where(i1 == g, -1.0, gm[g]) for g in range(NGROUP)]
    i2 = jnp.zeros((1, T), jnp.int32)
    v2 = gm2[0]
    for g in range(1, NGROUP):
        take = gm2[g] > v2
        i2 = jnp.where(take, g, i2)
        v2 = jnp.where(take, gm2[g], v2)
    mrows = [jnp.where((i1 == e // 2) | (i2 == e // 2), rows[e], 0.0)
             for e in range(E)]
    e1 = jnp.zeros((1, T), jnp.int32)
    w1 = mrows[0]
    for e in range(1, E):
        take = mrows[e] > w1
        e1 = jnp.where(take, e, e1)
        w1 = jnp.where(take, mrows[e], w1)
    mrows2 = [jnp.where(e1 == e, -1.0, mrows[e]) for e in range(E)]
    e2 = jnp.zeros((1, T), jnp.int32)
    w2 = mrows2[0]
    for e in range(1, E):
        take = mrows2[e] > w2
        e2 = jnp.where(take, e, e2)
        w2 = jnp.where(take, mrows2[e], w2)
    norm = w1 + w2 + 1e-20
    w1_ref[...] = w1 / norm
    w2_ref[...] = w2 / norm

    c = jnp.concatenate(
        [(e1 == e).astype(jnp.float32) + (e2 == e).astype(jnp.float32)
         for e in range(E)], axis=0)
    inc = c
    zeros = jnp.zeros((E, T), jnp.float32)
    for sh in (1, 2, 4, 8, 16, 32, 64, 128, 256, 512, 1024):
        shifted = jnp.concatenate([zeros[:, :sh], inc[:, :T - sh]], axis=1)
        inc = inc + shifted
    cb = inc - c
    cnt = jnp.sum(c, axis=1, keepdims=True).astype(jnp.int32)
    pad = ((cnt + (B - 1)) // B) * B
    offs = [jnp.zeros((1, 1), jnp.int32)]
    for e in range(1, E + 1):
        offs.append(offs[e - 1] + pad[e - 1:e, :])
    cbi = cb.astype(jnp.int32)
    p1 = sum(jnp.where(e1 == e, offs[e][0, 0] + cbi[e:e + 1, :], 0)
             for e in range(E))
    p2 = sum(jnp.where(e2 == e, offs[e][0, 0] + cbi[e:e + 1, :], 0)
             for e in range(E))
    p1_ref[...] = p1
    p2_ref[...] = p2
    bi = lax.broadcasted_iota(jnp.int32, (1, NRB), 1) * B
    bmap_ref[...] = sum((bi >= offs[e][0, 0]).astype(jnp.int32)
                        for e in range(1, E))


def _router_sort(x, gate_w):
    return pl.pallas_call(
        _router_body,
        out_shape=(
            jax.ShapeDtypeStruct((1, T), jnp.int32),
            jax.ShapeDtypeStruct((1, T), jnp.int32),
            jax.ShapeDtypeStruct((1, T), jnp.float32),
            jax.ShapeDtypeStruct((1, T), jnp.float32),
            jax.ShapeDtypeStruct((1, NRB), jnp.int32),
        ),
    )(x, gate_w)


_IC = 512


def _invert_body(p1_ref, p2_ref, w1_ref, w2_ref, src_ref, wv_ref):
    s = pl.program_id(0) * _IC
    rid = lax.broadcasted_iota(jnp.int32, (_IC, 1), 0) + s
    tok = lax.broadcasted_iota(jnp.int32, (1, T), 1)
    m1 = p1_ref[...] == rid
    m2 = p2_ref[...] == rid
    src_ref[...] = (jnp.sum(jnp.where(m1, tok, 0), axis=1, keepdims=True)
                    + jnp.sum(jnp.where(m2, tok, 0), axis=1, keepdims=True))
    wv_ref[...] = (jnp.sum(jnp.where(m1, w1_ref[...], 0.0), axis=1,
                           keepdims=True)
                   + jnp.sum(jnp.where(m2, w2_ref[...], 0.0), axis=1,
                             keepdims=True))


def _invert(p1, p2, w1, w2):
    return pl.pallas_call(
        _invert_body,
        grid=(CAP // _IC,),
        in_specs=[
            pl.BlockSpec((1, T), lambda b: (0, 0)),
            pl.BlockSpec((1, T), lambda b: (0, 0)),
            pl.BlockSpec((1, T), lambda b: (0, 0)),
            pl.BlockSpec((1, T), lambda b: (0, 0)),
        ],
        out_specs=(
            pl.BlockSpec((_IC, 1), lambda b: (b, 0)),
            pl.BlockSpec((_IC, 1), lambda b: (b, 0)),
        ),
        out_shape=(
            jax.ShapeDtypeStruct((CAP, 1), jnp.int32),
            jax.ShapeDtypeStruct((CAP, 1), jnp.float32),
        ),
    )(p1, p2, w1, w2)


def _make_sc_gather(nrows_out, ncols):
    info = plsc.get_sparse_core_info()
    nw = info.num_cores * info.num_subcores
    b_per_w = nrows_out // nw
    ch = 32
    assert b_per_w % ch == 0
    mesh = plsc.VectorSubcoreMesh(core_axis_name="c", subcore_axis_name="s")

    @functools.partial(
        pl.kernel,
        out_type=jax.ShapeDtypeStruct((nrows_out, ncols), jnp.float32),
        mesh=mesh,
        scratch_types=[
            pltpu.VMEM((ch,), jnp.int32),
            pltpu.VMEM((ch, ncols), jnp.float32),
            pltpu.SemaphoreType.DMA,
        ],
    )
    def gather_k(table_hbm, idx_hbm, out_hbm, idx_v, rows_v, sem):
        wid = lax.axis_index("s") * info.num_cores + lax.axis_index("c")
        base = wid * b_per_w
        for i in range(b_per_w // ch):
            b = base + i * ch
            pltpu.sync_copy(idx_hbm.at[pl.ds(b, ch)], idx_v)
            pltpu.async_copy(table_hbm.at[idx_v], rows_v, sem).wait()
            pltpu.sync_copy(rows_v, out_hbm.at[pl.ds(b, ch)])

    return gather_k


def _mm1_body(bm_ref, xs_ref, wg_ref, wu_ref, wv_ref, h1_ref):
    xs = xs_ref[...]
    g = lax.dot_general(xs, wg_ref[0], (((1,), (1,)), ((), ())),
                        preferred_element_type=jnp.float32)
    u = lax.dot_general(xs, wu_ref[0], (((1,), (1,)), ((), ())),
                        preferred_element_type=jnp.float32)
    h1_ref[...] = (g * jax.nn.sigmoid(g)) * u * wv_ref[...]


def _mm1(bmap, xs, w_gate_up, wv):
    grid_spec = pltpu.PrefetchScalarGridSpec(
        num_scalar_prefetch=1,
        grid=(NRB,),
        in_specs=[
            pl.BlockSpec((B, D), lambda b, bm: (b, 0)),
            pl.BlockSpec((1, DFF, D), lambda b, bm: (bm[b], 0, 0)),
            pl.BlockSpec((1, DFF, D), lambda b, bm: (bm[b], 1, 0)),
            pl.BlockSpec((B, 1), lambda b, bm: (b, 0)),
        ],
        out_specs=pl.BlockSpec((B, DFF), lambda b, bm: (b, 0)),
    )
    return pl.pallas_call(
        _mm1_body,
        grid_spec=grid_spec,
        out_shape=jax.ShapeDtypeStruct((CAP, DFF), jnp.float32),
    )(bmap, xs, w_gate_up, w_gate_up, wv)


def _mm2_body(bm_ref, h1_ref, wd_ref, outd_ref):
    outd_ref[...] = lax.dot_general(h1_ref[...], wd_ref[0],
                                    (((1,), (1,)), ((), ())),
                                    preferred_element_type=jnp.float32)


def _mm2(bmap, h1, w_down):
    grid_spec = pltpu.PrefetchScalarGridSpec(
        num_scalar_prefetch=1,
        grid=(NRB,),
        in_specs=[
            pl.BlockSpec((B, DFF), lambda b, bm: (b, 0)),
            pl.BlockSpec((1, D, DFF), lambda b, bm: (bm[b], 0, 0)),
        ],
        out_specs=pl.BlockSpec((B, D), lambda b, bm: (b, 0)),
    )
    return pl.pallas_call(
        _mm2_body,
        grid_spec=grid_spec,
        out_shape=jax.ShapeDtypeStruct((CAP, D), jnp.float32),
    )(bmap, h1, w_down)


_TB = 128
_NTB = T // _TB


def _mm1s_body(x_ref, sg_ref, su_ref, h1s_ref):
    x = x_ref[...]
    g = lax.dot_general(x, sg_ref[...], (((1,), (1,)), ((), ())),
                        preferred_element_type=jnp.float32)
    u = lax.dot_general(x, su_ref[...], (((1,), (1,)), ((), ())),
                        preferred_element_type=jnp.float32)
    h1s_ref[...] = (g * jax.nn.sigmoid(g)) * u


def _mm1s(x, shared_gate_up):
    return pl.pallas_call(
        _mm1s_body,
        grid=(2, _NTB),
        in_specs=[
            pl.BlockSpec((_TB, D), lambda p, r: (r, 0)),
            pl.BlockSpec((DFF, D), lambda p, r: (p, 0)),
            pl.BlockSpec((DFF, D), lambda p, r: (p + 2, 0)),
        ],
        out_specs=pl.BlockSpec((_TB, DFF), lambda p, r: (p * _NTB + r, 0)),
        out_shape=jax.ShapeDtypeStruct((2 * T, DFF), jnp.float32),
    )(x, shared_gate_up, shared_gate_up)


def _mm2s_body(h1s_ref, sd_ref, g0_ref, g1_ref, out_ref):
    part = pl.program_id(1)
    acc = lax.dot_general(h1s_ref[...], sd_ref[...], (((1,), (1,)), ((), ())),
                          preferred_element_type=jnp.float32)

    @pl.when(part == 0)
    def _():
        out_ref[...] = acc + (g0_ref[...] + g1_ref[...]) * SCALE

    @pl.when(part != 0)
    def _():
        out_ref[...] += acc


def _mm2s_final(h1s, shared_down, g):
    return pl.pallas_call(
        _mm2s_body,
        grid=(_NTB, 2),
        in_specs=[
            pl.BlockSpec((_TB, DFF), lambda r, p: (p * _NTB + r, 0)),
            pl.BlockSpec((D, DFF), lambda r, p: (0, p)),
            pl.BlockSpec((_TB, D), lambda r, p: (r, 0)),
            pl.BlockSpec((_TB, D), lambda r, p: (_NTB + r, 0)),
        ],
        out_specs=pl.BlockSpec((_TB, D), lambda r, p: (r, 0)),
        out_shape=jax.ShapeDtypeStruct((T, D), jnp.float32),
    )(h1s, shared_down, g, g)


def kernel(hidden_states, gate_w, w_gate_up, w_down, shared_gate_up,
           shared_down):
    x = hidden_states
    p1, p2, w1, w2, bmap = _router_sort(x, gate_w)
    src, wv = _invert(p1, p2, w1, w2)
    bmap1 = jnp.reshape(bmap, (NRB,))
    src1 = jnp.reshape(src, (CAP,))
    xs = _make_sc_gather(CAP, D)(x, src1)
    h1 = _mm1(bmap1, xs, w_gate_up, wv)
    outd = _mm2(bmap1, h1, w_down)
    pc = jnp.concatenate([jnp.reshape(p1, (T,)), jnp.reshape(p2, (T,))])
    g = _make_sc_gather(2 * T, D)(outd, pc)
    h1s = _mm1s(x, shared_gate_up)
    return _mm2s_final(h1s, shared_down, g)


# speedup vs baseline: 1.2439x; 1.2439x over previous
import functools

import jax
import jax.numpy as jnp
from jax import lax
from jax.experimental import pallas as pl
from jax.experimental.pallas import tpu as pltpu
from jax.experimental.pallas import tpu_sc as plsc

T = 2048
D = 2048
DFF = 1408
E = 8
NGROUP = 4
B = 256
CAP = 2 * T + E * B
NRB = CAP // B
SCALE = 1.0


def _router_body(x_ref, gw_ref, p1_ref, p2_ref, w1_ref, w2_ref, bmap_ref):
    logits = lax.dot_general(gw_ref[...], x_ref[...], (((1,), (1,)), ((), ())),
                             preferred_element_type=jnp.float32)
    m = jnp.max(logits, axis=0, keepdims=True)
    ex = jnp.exp(logits - m)
    sc = ex / jnp.sum(ex, axis=0, keepdims=True)

    rows = [sc[e:e + 1, :] for e in range(E)]
    gm = [jnp.maximum(rows[2 * g], rows[2 * g + 1]) for g in range(NGROUP)]
    i1 = jnp.zeros((1, T), jnp.int32)
    v1 = gm[0]
    for g in range(1, NGROUP):
        take = gm[g] > v1
        i1 = jnp.where(take, g, i1)
        v1 = jnp.where(take, gm[g], v1)
    gm2 = [jnp.where(i1 == g, -1.0, gm[g]) for g in range(NGROUP)]
    i2 = jnp.zeros((1, T), jnp.int32)
    v2 = gm2[0]
    for g in range(1, NGROUP):
        take = gm2[g] > v2
        i2 = jnp.where(take, g, i2)
        v2 = jnp.where(take, gm2[g], v2)
    mrows = [jnp.where((i1 == e // 2) | (i2 == e // 2), rows[e], 0.0)
             for e in range(E)]
    e1 = jnp.zeros((1, T), jnp.int32)
    w1 = mrows[0]
    for e in range(1, E):
        take = mrows[e] > w1
        e1 = jnp.where(take, e, e1)
        w1 = jnp.where(take, mrows[e], w1)
    mrows2 = [jnp.where(e1 == e, -1.0, mrows[e]) for e in range(E)]
    e2 = jnp.zeros((1, T), jnp.int32)
    w2 = mrows2[0]
    for e in range(1, E):
        take = mrows2[e] > w2
        e2 = jnp.where(take, e, e2)
        w2 = jnp.where(take, mrows2[e], w2)
    norm = w1 + w2 + 1e-20
    w1_ref[...] = w1 / norm
    w2_ref[...] = w2 / norm

    c = jnp.concatenate(
        [(e1 == e).astype(jnp.float32) + (e2 == e).astype(jnp.float32)
         for e in range(E)], axis=0)
    inc = c
    zeros = jnp.zeros((E, T), jnp.float32)
    for sh in (1, 2, 4, 8, 16, 32, 64, 128, 256, 512, 1024):
        shifted = jnp.concatenate([zeros[:, :sh], inc[:, :T - sh]], axis=1)
        inc = inc + shifted
    cb = inc - c
    cnt = jnp.sum(c, axis=1, keepdims=True).astype(jnp.int32)
    pad = ((cnt + (B - 1)) // B) * B
    offs = [jnp.zeros((1, 1), jnp.int32)]
    for e in range(1, E + 1):
        offs.append(offs[e - 1] + pad[e - 1:e, :])
    cbi = cb.astype(jnp.int32)
    p1 = sum(jnp.where(e1 == e, offs[e][0, 0] + cbi[e:e + 1, :], 0)
             for e in range(E))
    p2 = sum(jnp.where(e2 == e, offs[e][0, 0] + cbi[e:e + 1, :], 0)
             for e in range(E))
    p1_ref[...] = p1
    p2_ref[...] = p2
    bi = lax.broadcasted_iota(jnp.int32, (1, NRB), 1) * B
    bmap_ref[...] = sum((bi >= offs[e][0, 0]).astype(jnp.int32)
                        for e in range(1, E))


def _router_sort(x, gate_w):
    return pl.pallas_call(
        _router_body,
        out_shape=(
            jax.ShapeDtypeStruct((1, T), jnp.int32),
            jax.ShapeDtypeStruct((1, T), jnp.int32),
            jax.ShapeDtypeStruct((1, T), jnp.float32),
            jax.ShapeDtypeStruct((1, T), jnp.float32),
            jax.ShapeDtypeStruct((1, NRB), jnp.int32),
        ),
    )(x, gate_w)


_IC = 512


def _invert_body(p1_ref, p2_ref, w1_ref, w2_ref, src_ref, wv_ref):
    s = pl.program_id(0) * _IC
    rid = lax.broadcasted_iota(jnp.int32, (_IC, 1), 0) + s
    tok = lax.broadcasted_iota(jnp.int32, (1, T), 1)
    m1 = p1_ref[...] == rid
    m2 = p2_ref[...] == rid
    src_ref[...] = (jnp.sum(jnp.where(m1, tok, 0), axis=1, keepdims=True)
                    + jnp.sum(jnp.where(m2, tok, 0), axis=1, keepdims=True))
    wv_ref[...] = (jnp.sum(jnp.where(m1, w1_ref[...], 0.0), axis=1,
                           keepdims=True)
                   + jnp.sum(jnp.where(m2, w2_ref[...], 0.0), axis=1,
                             keepdims=True))


def _invert(p1, p2, w1, w2):
    return pl.pallas_call(
        _invert_body,
        grid=(CAP // _IC,),
        in_specs=[
            pl.BlockSpec((1, T), lambda b: (0, 0)),
            pl.BlockSpec((1, T), lambda b: (0, 0)),
            pl.BlockSpec((1, T), lambda b: (0, 0)),
            pl.BlockSpec((1, T), lambda b: (0, 0)),
        ],
        out_specs=(
            pl.BlockSpec((_IC, 1), lambda b: (b, 0)),
            pl.BlockSpec((_IC, 1), lambda b: (b, 0)),
        ),
        out_shape=(
            jax.ShapeDtypeStruct((CAP, 1), jnp.int32),
            jax.ShapeDtypeStruct((CAP, 1), jnp.float32),
        ),
    )(p1, p2, w1, w2)


def _make_sc_gather(nrows_out, ncols):
    info = plsc.get_sparse_core_info()
    nw = info.num_cores * info.num_subcores
    b_per_w = nrows_out // nw
    ch = 16
    nch = b_per_w // ch
    assert b_per_w % ch == 0
    mesh = plsc.VectorSubcoreMesh(core_axis_name="c", subcore_axis_name="s")

    @functools.partial(
        pl.kernel,
        out_type=jax.ShapeDtypeStruct((nrows_out, ncols), jnp.float32),
        mesh=mesh,
        scratch_types=[
            pltpu.VMEM((b_per_w,), jnp.int32),
            pltpu.VMEM((ch, ncols), jnp.float32),
            pltpu.VMEM((ch, ncols), jnp.float32),
            pltpu.SemaphoreType.DMA,
            pltpu.SemaphoreType.DMA,
            pltpu.SemaphoreType.DMA,
            pltpu.SemaphoreType.DMA,
        ],
    )
    def gather_k(table_hbm, idx_hbm, out_hbm, idx_v, rows0, rows1,
                 gs0, gs1, ws0, ws1):
        wid = lax.axis_index("s") * info.num_cores + lax.axis_index("c")
        base = wid * b_per_w
        pltpu.sync_copy(idx_hbm.at[pl.ds(base, b_per_w)], idx_v)
        rows = (rows0, rows1)
        gsem = (gs0, gs1)
        wsem = (ws0, ws1)
        hg = [None, None]
        hw = [None, None]
        hg[0] = pltpu.make_async_copy(
            table_hbm.at[idx_v.at[pl.ds(0, ch)]], rows[0], gsem[0])
        hg[0].start()
        for i in range(nch):
            j = i % 2
            jn = (i + 1) % 2
            if i + 1 < nch:
                if hw[jn] is not None:
                    hw[jn].wait()
                    hw[jn] = None
                hg[jn] = pltpu.make_async_copy(
                    table_hbm.at[idx_v.at[pl.ds((i + 1) * ch, ch)]],
                    rows[jn], gsem[jn])
                hg[jn].start()
            hg[j].wait()
            hw[j] = pltpu.make_async_copy(
                rows[j], out_hbm.at[pl.ds(base + i * ch, ch)], wsem[j])
            hw[j].start()
        for j in range(2):
            if hw[j] is not None:
                hw[j].wait()

    return gather_k


def _mm1_body(bm_ref, xs_ref, wg_ref, wu_ref, wv_ref, h1_ref):
    xs = xs_ref[...]
    g = lax.dot_general(xs, wg_ref[0], (((1,), (1,)), ((), ())),
                        preferred_element_type=jnp.float32)
    u = lax.dot_general(xs, wu_ref[0], (((1,), (1,)), ((), ())),
                        preferred_element_type=jnp.float32)
    h1_ref[...] = (g * jax.nn.sigmoid(g)) * u * wv_ref[...]


def _mm1(bmap, xs, w_gate_up, wv):
    grid_spec = pltpu.PrefetchScalarGridSpec(
        num_scalar_prefetch=1,
        grid=(NRB,),
        in_specs=[
            pl.BlockSpec((B, D), lambda b, bm: (b, 0)),
            pl.BlockSpec((1, DFF, D), lambda b, bm: (bm[b], 0, 0)),
            pl.BlockSpec((1, DFF, D), lambda b, bm: (bm[b], 1, 0)),
            pl.BlockSpec((B, 1), lambda b, bm: (b, 0)),
        ],
        out_specs=pl.BlockSpec((B, DFF), lambda b, bm: (b, 0)),
    )
    return pl.pallas_call(
        _mm1_body,
        grid_spec=grid_spec,
        out_shape=jax.ShapeDtypeStruct((CAP, DFF), jnp.float32),
    )(bmap, xs, w_gate_up, w_gate_up, wv)


def _mm2_body(bm_ref, h1_ref, wd_ref, outd_ref):
    outd_ref[...] = lax.dot_general(h1_ref[...], wd_ref[0],
                                    (((1,), (1,)), ((), ())),
                                    preferred_element_type=jnp.float32)


def _mm2(bmap, h1, w_down):
    grid_spec = pltpu.PrefetchScalarGridSpec(
        num_scalar_prefetch=1,
        grid=(NRB,),
        in_specs=[
            pl.BlockSpec((B, DFF), lambda b, bm: (b, 0)),
            pl.BlockSpec((1, D, DFF), lambda b, bm: (bm[b], 0, 0)),
        ],
        out_specs=pl.BlockSpec((B, D), lambda b, bm: (b, 0)),
    )
    return pl.pallas_call(
        _mm2_body,
        grid_spec=grid_spec,
        out_shape=jax.ShapeDtypeStruct((CAP, D), jnp.float32),
    )(bmap, h1, w_down)


_TB = 256
_NTB = T // _TB


def _mm1s_body(x_ref, sg_ref, su_ref, h1s_ref):
    x = x_ref[...]
    g = lax.dot_general(x, sg_ref[...], (((1,), (1,)), ((), ())),
                        preferred_element_type=jnp.float32)
    u = lax.dot_general(x, su_ref[...], (((1,), (1,)), ((), ())),
                        preferred_element_type=jnp.float32)
    h1s_ref[...] = (g * jax.nn.sigmoid(g)) * u


def _mm1s(x, shared_gate_up):
    return pl.pallas_call(
        _mm1s_body,
        grid=(2, _NTB),
        in_specs=[
            pl.BlockSpec((_TB, D), lambda p, r: (r, 0)),
            pl.BlockSpec((DFF, D), lambda p, r: (p, 0)),
            pl.BlockSpec((DFF, D), lambda p, r: (p + 2, 0)),
        ],
        out_specs=pl.BlockSpec((_TB, DFF), lambda p, r: (p * _NTB + r, 0)),
        out_shape=jax.ShapeDtypeStruct((2 * T, DFF), jnp.float32),
    )(x, shared_gate_up, shared_gate_up)


def _mm2s_body(h1s_ref, sd_ref, out_ref):
    part = pl.program_id(1)
    acc = lax.dot_general(h1s_ref[...], sd_ref[...], (((1,), (1,)), ((), ())),
                          preferred_element_type=jnp.float32)

    @pl.when(part == 0)
    def _():
        out_ref[...] = acc

    @pl.when(part != 0)
    def _():
        out_ref[...] += acc


def _mm2s(h1s, shared_down):
    return pl.pallas_call(
        _mm2s_body,
        grid=(_NTB, 2),
        in_specs=[
            pl.BlockSpec((_TB, DFF), lambda r, p: (p * _NTB + r, 0)),
            pl.BlockSpec((D, DFF), lambda r, p: (0, p)),
        ],
        out_specs=pl.BlockSpec((_TB, D), lambda r, p: (r, 0)),
        out_shape=jax.ShapeDtypeStruct((T, D), jnp.float32),
    )(h1s, shared_down)


def _final_body(sh_ref, g0_ref, g1_ref, out_ref):
    out_ref[...] = sh_ref[...] + (g0_ref[...] + g1_ref[...]) * SCALE


def _final_add(sh, g):
    return pl.pallas_call(
        _final_body,
        grid=(_NTB,),
        in_specs=[
            pl.BlockSpec((_TB, D), lambda r: (r, 0)),
            pl.BlockSpec((_TB, D), lambda r: (r, 0)),
            pl.BlockSpec((_TB, D), lambda r: (_NTB + r, 0)),
        ],
        out_specs=pl.BlockSpec((_TB, D), lambda r: (r, 0)),
        out_shape=jax.ShapeDtypeStruct((T, D), jnp.float32),
    )(sh, g, g)


def kernel(hidden_states, gate_w, w_gate_up, w_down, shared_gate_up,
           shared_down):
    x = hidden_states
    p1, p2, w1, w2, bmap = _router_sort(x, gate_w)
    src, wv = _invert(p1, p2, w1, w2)
    bmap1 = jnp.reshape(bmap, (NRB,))
    src1 = jnp.reshape(src, (CAP,))
    xs = _make_sc_gather(CAP, D)(x, src1)
    h1s = _mm1s(x, shared_gate_up)
    h1 = _mm1(bmap1, xs, w_gate_up, wv)
    outd = _mm2(bmap1, h1, w_down)
    pc = jnp.concatenate([jnp.reshape(p1, (T,)), jnp.reshape(p2, (T,))])
    g = _make_sc_gather(2 * T, D)(outd, pc)
    sh = _mm2s(h1s, shared_down)
    return _final_add(sh, g)
